# baseline (speedup 1.0000x reference)
import functools

import jax
import jax.numpy as jnp
from jax import lax
from jax.experimental import pallas as pl
from jax.experimental.pallas import tpu as pltpu
from jax.experimental.pallas import tpu_sc as plsc

_NC = 2
_NS = 16
_NW = _NC * _NS

_CIMG = 2
_NSLOT = 3
_NDRV = 2
_RA = 2
_N_SP = 144
_N_TI = 112
_TI_TILES = 14
_TI_PER = _N_TI // _TI_TILES
_HALF = 64


def _ring(chunks, read, wait_read, write, wait_write):
    n = len(chunks)
    ra = min(_RA, n)
    for k in range(ra):
        read(chunks[k], k % _NSLOT)
    for k in range(n):
        slot = k % _NSLOT
        wait_read(chunks[k], slot)
        write(chunks[k], slot)
        nk = k + ra
        if nk < n:
            conflict = nk - _NSLOT
            if conflict >= 0:
                wait_write(chunks[conflict], conflict % _NSLOT)
            read(chunks[nk], nk % _NSLOT)
    for k in range(max(0, n - _NSLOT), n):
        wait_write(chunks[k], k % _NSLOT)


def _sc_body(n_imgs, in_hbm, out_hbm, sbufs, srs, sws, tbufs, trs, tws):
    c = lax.axis_index("c")
    s = lax.axis_index("s")
    sc_base = c * n_imgs

    n_sp_chunks = _N_SP // _CIMG
    for d in range(_NDRV):
        @pl.when(s == d)
        def _(d=d):
            b0 = d * _NSLOT

            def read(g, slot):
                pltpu.async_copy(
                    in_hbm.at[pl.ds(sc_base + g * _CIMG, _CIMG)],
                    sbufs[b0 + slot], srs[b0 + slot])

            def wait_read(g, slot):
                pltpu.make_async_copy(
                    in_hbm.at[pl.ds(sc_base + g * _CIMG, _CIMG)],
                    sbufs[b0 + slot], srs[b0 + slot]).wait()

            def write(g, slot):
                pltpu.async_copy(
                    sbufs[b0 + slot],
                    out_hbm.at[pl.ds(sc_base + g * _CIMG, _CIMG)],
                    sws[b0 + slot])

            def wait_write(g, slot):
                pltpu.make_async_copy(
                    sbufs[b0 + slot],
                    out_hbm.at[pl.ds(sc_base + g * _CIMG, _CIMG)],
                    sws[b0 + slot]).wait()

            _ring(list(range(d, n_sp_chunks, _NDRV)),
                  read, wait_read, write, wait_write)

    @pl.when(s >= _NDRV)
    def _():
        t = s - _NDRV
        img_base = sc_base + _N_SP + t * _TI_PER

        def src(ch):
            img, half = ch
            return in_hbm.at[pl.ds(img_base + img, 1),
                             pl.ds(half * _HALF, _HALF)]

        def dst(ch):
            img, half = ch
            return out_hbm.at[pl.ds(img_base + img, 1),
                              pl.ds(half * _HALF, _HALF)]

        def read(ch, slot):
            pltpu.async_copy(src(ch), tbufs[slot], trs[slot])

        def wait_read(ch, slot):
            pltpu.make_async_copy(src(ch), tbufs[slot], trs[slot]).wait()

        def write(ch, slot):
            pltpu.async_copy(tbufs[slot], dst(ch), tws[slot])

        def wait_write(ch, slot):
            pltpu.make_async_copy(tbufs[slot], dst(ch), tws[slot]).wait()

        chunks = [(i, h) for i in range(_TI_PER) for h in range(256 // _HALF)]
        _ring(chunks, read, wait_read, write, wait_write)


def kernel(phi):
    B, H, W = phi.shape
    n_imgs = B // _NC
    assert n_imgs == _N_SP + _N_TI and _N_SP % _CIMG == 0

    mesh = plsc.VectorSubcoreMesh(core_axis_name="c", subcore_axis_name="s")
    run = pl.kernel(
        functools.partial(_sc_body, n_imgs),
        mesh=mesh,
        out_type=jax.ShapeDtypeStruct(phi.shape, phi.dtype),
        scratch_types=[
            [pltpu.VMEM_SHARED((_CIMG, H, W), jnp.float32)] * (_NSLOT * _NDRV),
            [pltpu.SemaphoreType.DMA] * (_NSLOT * _NDRV),
            [pltpu.SemaphoreType.DMA] * (_NSLOT * _NDRV),
            [pltpu.VMEM((1, _HALF, W), jnp.float32)] * _NSLOT,
            [pltpu.SemaphoreType.DMA] * _NSLOT,
            [pltpu.SemaphoreType.DMA] * _NSLOT,
        ],
    )
    return run(phi)


# speedup vs baseline: 1.0329x; 1.0329x over previous
import functools

import jax
import jax.numpy as jnp
from jax import lax
from jax.experimental import pallas as pl
from jax.experimental.pallas import tpu as pltpu
from jax.experimental.pallas import tpu_sc as plsc

_NC = 2
_NS = 16
_NW = _NC * _NS

_CIMG = 4
_NSLOT = 3
_NDRV = 2
_RA = 2


def _sc_body(n_chunks, in_hbm, out_hbm, bufs, rsems, wsems):
    c = lax.axis_index("c")
    s = lax.axis_index("s")

    for d in range(_NDRV):
        @pl.when(s == d)
        def _(d=d):
            chunks = list(range(d, n_chunks, _NDRV))

            def img0(g):
                return (c * n_chunks + g) * _CIMG

            def read(g, slot):
                pltpu.async_copy(
                    in_hbm.at[pl.ds(img0(g), _CIMG)], bufs[slot], rsems[slot])

            def wait_read(g, slot):
                pltpu.make_async_copy(
                    in_hbm.at[pl.ds(img0(g), _CIMG)], bufs[slot],
                    rsems[slot]).wait()

            def write(g, slot):
                pltpu.async_copy(
                    bufs[slot], out_hbm.at[pl.ds(img0(g), _CIMG)], wsems[slot])

            def wait_write(g, slot):
                pltpu.make_async_copy(
                    bufs[slot], out_hbm.at[pl.ds(img0(g), _CIMG)],
                    wsems[slot]).wait()

            base_slot = d * _NSLOT
            n = len(chunks)
            ra = min(_RA, n)
            for k in range(ra):
                read(chunks[k], base_slot + k % _NSLOT)
            for k in range(n):
                slot = base_slot + k % _NSLOT
                wait_read(chunks[k], slot)
                write(chunks[k], slot)
                nk = k + ra
                if nk < n:
                    conflict = nk - _NSLOT
                    if conflict >= 0:
                        wait_write(chunks[conflict],
                                   base_slot + conflict % _NSLOT)
                    read(chunks[nk], base_slot + nk % _NSLOT)
            for k in range(max(0, n - _NSLOT), n):
                wait_write(chunks[k], base_slot + k % _NSLOT)


def kernel(phi):
    B, H, W = phi.shape
    assert B % (_NC * _CIMG) == 0
    n_chunks = B // (_NC * _CIMG)

    mesh = plsc.VectorSubcoreMesh(core_axis_name="c", subcore_axis_name="s")
    run = pl.kernel(
        functools.partial(_sc_body, n_chunks),
        mesh=mesh,
        out_type=jax.ShapeDtypeStruct(phi.shape, phi.dtype),
        scratch_types=[
            [pltpu.VMEM_SHARED((_CIMG, H, W), jnp.float32)] * (_NSLOT * _NDRV),
            [pltpu.SemaphoreType.DMA] * (_NSLOT * _NDRV),
            [pltpu.SemaphoreType.DMA] * (_NSLOT * _NDRV),
        ],
    )
    return run(phi)
